# baseline (device time: 13367 ns/iter reference)
import jax
import jax.numpy as jnp
from jax import lax
from jax.experimental import pallas as pl
from jax.experimental.pallas import tpu as pltpu

Z_DIM = 4
N_CHUNKS = 4


def kernel(x, pi):
    shard_shape = x.shape
    n_rows = shard_shape[1]
    scale_shape = (shard_shape[0], n_rows)

    def body(
        x_ref,
        pi_ref,
        out_ref,
        send_q,
        send_s,
        recv_q,
        recv_s,
        send_sems,
        recv_sems,
        scale_send_sems,
        scale_recv_sems,
    ):
        my_x = lax.axis_index("x")
        my_y = lax.axis_index("y")
        my_z = lax.axis_index("z")

        dst_z = pi_ref[my_z]
        src_z = jnp.int32(0)
        for s in range(Z_DIM):
            src_z = jnp.where(pi_ref[s] == my_z, jnp.int32(s), src_z)

        barrier_sem = pltpu.get_barrier_semaphore()
        for nbr_z in (dst_z, src_z):
            pl.semaphore_signal(
                barrier_sem,
                inc=1,
                device_id=(my_x, my_y, nbr_z),
                device_id_type=pl.DeviceIdType.MESH,
            )

        chunk = n_rows // N_CHUNKS
        rdmas = []
        for c in range(N_CHUNKS):
            rows = pl.ds(c * chunk, chunk)
            xv = x_ref[:, rows, :]
            amax = jnp.maximum(jnp.max(jnp.abs(xv), axis=-1), 1e-30)
            send_s[:, rows] = amax * (1.0 / 127.0)
            inv = (127.0 / amax)[:, :, None]
            send_q[:, rows, :] = jnp.rint(xv * inv).astype(jnp.int8)

            if c == 0:
                pl.semaphore_wait(barrier_sem, 2)

            data = pltpu.make_async_remote_copy(
                src_ref=send_q.at[:, rows, :],
                dst_ref=recv_q.at[:, rows, :],
                send_sem=send_sems.at[c],
                recv_sem=recv_sems.at[c],
                device_id=(my_x, my_y, dst_z),
                device_id_type=pl.DeviceIdType.MESH,
            )
            data.start()
            scales = pltpu.make_async_remote_copy(
                src_ref=send_s.at[:, rows],
                dst_ref=recv_s.at[:, rows],
                send_sem=scale_send_sems.at[c],
                recv_sem=scale_recv_sems.at[c],
                device_id=(my_x, my_y, dst_z),
                device_id_type=pl.DeviceIdType.MESH,
            )
            scales.start()
            rdmas.append((data, scales))

        for c, (data, scales) in enumerate(rdmas):
            rows = pl.ds(c * chunk, chunk)
            scales.wait_recv()
            data.wait_recv()
            out_ref[:, rows, :] = (
                recv_q[:, rows, :].astype(jnp.float32)
                * recv_s[:, rows][:, :, None]
            ).astype(jnp.bfloat16)

        for data, scales in rdmas:
            data.wait_send()
            scales.wait_send()

    return pl.pallas_call(
        body,
        out_shape=jax.ShapeDtypeStruct(shard_shape, jnp.bfloat16),
        in_specs=[
            pl.BlockSpec(memory_space=pltpu.VMEM),
            pl.BlockSpec(memory_space=pltpu.SMEM),
        ],
        out_specs=pl.BlockSpec(memory_space=pltpu.VMEM),
        scratch_shapes=[
            pltpu.VMEM(shard_shape, jnp.int8),
            pltpu.VMEM(scale_shape, jnp.float32),
            pltpu.VMEM(shard_shape, jnp.int8),
            pltpu.VMEM(scale_shape, jnp.float32),
            pltpu.SemaphoreType.DMA((N_CHUNKS,)),
            pltpu.SemaphoreType.DMA((N_CHUNKS,)),
            pltpu.SemaphoreType.DMA((N_CHUNKS,)),
            pltpu.SemaphoreType.DMA((N_CHUNKS,)),
        ],
        compiler_params=pltpu.CompilerParams(collective_id=0),
    )(x, pi)
